# baseline (speedup 1.0000x reference)
import functools

import jax
import jax.numpy as jnp
from jax import lax
from jax.experimental import pallas as pl
from jax.experimental.pallas import tpu as pltpu
from jax.experimental.pallas import tpu_sc as plsc

NF = 10000
NE = 160000
H = 256
HH = 128
NCORE = 2
NSUB = 16
K = 128
NCH_REL = 79
EPT = NCH_REL * K
EPAD = EPT * NSUB
NCH = 2 * NCH_REL + 2
NCH_PAD = NCH + 4
ROWS_PER_TILE = 632
R1 = 1000


def _dense1_body(xf, xa, wpf, bpf, wpa, bpa, wr1, wr2, wq1, wq2,
                 br1, bq1, br2, bq2, tab_o, base_o):
    hf = jnp.dot(xf[...], wpf[...], preferred_element_type=jnp.float32) + bpf[...]
    ha = jnp.dot(xa[...], wpa[...], preferred_element_type=jnp.float32) + bpa[...]
    tf = jnp.dot(hf, wr2[...], preferred_element_type=jnp.float32) * 0.5
    ta = jnp.dot(ha, wr1[...], preferred_element_type=jnp.float32) * 0.5
    wroot = (wq1[...] + wq2[...]) * 0.5
    bias = (br1[...] + bq1[...] + br2[...] + bq2[...]) * 0.5
    base = jnp.dot(hf, wroot, preferred_element_type=jnp.float32) + bias
    tab_o[0] = ta[:, :HH]
    tab_o[1] = tf[:, :HH]
    tab_o[2] = ta[:, HH:]
    tab_o[3] = tf[:, HH:]
    base_o[0] = base[:, :HH]
    base_o[1] = base[:, HH:]


def _dense1(x_flight, x_airport, wpf, bpf, wpa, bpa, wr1, wr2, wq1, wq2,
            br1, bq1, br2, bq2):
    n_blocks = NF // R1
    full = lambda shape: pl.BlockSpec(shape, lambda i: (0,) * len(shape))
    row_blk = lambda d: pl.BlockSpec((R1, d), lambda i: (i, 0))
    return pl.pallas_call(
        _dense1_body,
        grid=(n_blocks,),
        in_specs=[
            row_blk(256), row_blk(128),
            full((256, H)), full((H,)), full((128, H)), full((H,)),
            full((H, H)), full((H, H)), full((H, H)), full((H, H)),
            full((H,)), full((H,)), full((H,)), full((H,)),
        ],
        out_specs=[
            pl.BlockSpec((4, R1, HH), lambda i: (0, i, 0)),
            pl.BlockSpec((NCORE, R1, HH), lambda i: (0, i, 0)),
        ],
        out_shape=[
            jax.ShapeDtypeStruct((4, NF, HH), jnp.float32),
            jax.ShapeDtypeStruct((NCORE, NF, HH), jnp.float32),
        ],
    )(x_flight, x_airport, wpf, bpf, wpa, bpa, wr1, wr2, wq1, wq2,
      br1, bq1, br2, bq2)


def _sc_segment_add(tab, base2, src1, dst1):
    mesh = plsc.VectorSubcoreMesh(core_axis_name="c", subcore_axis_name="s")

    @functools.partial(
        pl.kernel,
        out_type=jax.ShapeDtypeStruct((NCORE * NF, HH), jnp.float32),
        mesh=mesh,
        scratch_types=[
            pltpu.VMEM_SHARED((NF + 16, HH), jnp.float32),
            [pltpu.VMEM((K,), jnp.int32)] * 4,
            [pltpu.VMEM((K,), jnp.int32)] * 4,
            pltpu.VMEM((K,), jnp.int32),
            [pltpu.VMEM((K, HH), jnp.float32)] * 2,
            [pltpu.SemaphoreType.DMA] * 4,
            [pltpu.SemaphoreType.DMA] * 2,
            [pltpu.SemaphoreType.DMA] * 2,
        ],
    )
    def sc_kernel(tab_h, base_h, src_h, dst_h, out_h,
                  acc_s, srcb, dstb, padd, rows, isem, gsem, ssem):
        cid = lax.axis_index("c")
        sid = lax.axis_index("s")
        row0 = cid * NF
        win = tab_h.at[pl.ds(pl.multiple_of(cid * (2 * NF), 8), 2 * NF)]
        tile_lo = pl.multiple_of(
            jnp.minimum(sid * ROWS_PER_TILE, NF - ROWS_PER_TILE), 8)

        def coff(c):
            return pl.multiple_of((sid * NCH_PAD + c) * K, K)

        def idx_start(c, s):
            pltpu.async_copy(src_h.at[pl.ds(coff(c), K)], srcb[s], isem[s])
            pltpu.async_copy(dst_h.at[pl.ds(coff(c), K)], dstb[s], isem[s])

        def idx_wait(s):
            pltpu.make_async_copy(src_h.at[pl.ds(0, K)], srcb[s], isem[s]).wait()
            pltpu.make_async_copy(dst_h.at[pl.ds(0, K)], dstb[s], isem[s]).wait()

        def gather_start(s, rb):
            pltpu.async_copy(win.at[srcb[s]], rows[rb], gsem[rb])

        def gather_wait(rb):
            pltpu.make_async_copy(win.at[srcb[0]], rows[rb], gsem[rb]).wait()

        def scatter_start(s, rb):
            pltpu.async_copy(rows[rb], acc_s.at[dstb[s]], ssem[rb], add=True)

        def scatter_wait(rb):
            pltpu.make_async_copy(rows[rb], acc_s.at[dstb[0]], ssem[rb]).wait()

        pltpu.sync_copy(base_h.at[pl.ds(row0 + tile_lo, ROWS_PER_TILE)],
                        acc_s.at[pl.ds(tile_lo, ROWS_PER_TILE)])
        for c in range(3):
            idx_start(c, c)
        pltpu.sync_copy(dst_h.at[pl.ds(coff(NCH), K)], padd)
        idx_wait(0)
        gather_start(0, 0)
        plsc.subcore_barrier()
        pltpu.async_copy(rows[1], acc_s.at[padd], ssem[1], add=True)

        def step(c, k):
            rb, rb1 = k % 2, (k + 1) % 2
            s_c, s_n, s_f = k % 4, (k + 1) % 4, (k + 3) % 4
            gather_wait(rb)
            scatter_start(s_c, rb)
            idx_wait(s_n)
            scatter_wait(rb1)
            gather_start(s_n, rb1)
            idx_start(c + 3, s_f)

        def quad(i, carry):
            for k in range(4):
                step(4 * i + k, k)
            return carry

        lax.fori_loop(0, NCH // 4, quad, 0)
        gather_wait(NCH % 2)
        scatter_wait((NCH - 1) % 2)
        idx_wait((NCH + 1) % 4)
        idx_wait((NCH + 2) % 4)
        plsc.subcore_barrier()
        pltpu.sync_copy(acc_s.at[pl.ds(tile_lo, ROWS_PER_TILE)],
                        out_h.at[pl.ds(row0 + tile_lo, ROWS_PER_TILE)])

    return sc_kernel(tab, base2, src1, dst1)


def _dense2_body(acc, wout, bout, out):
    a0 = jnp.maximum(acc[0], 0.0)
    a1 = jnp.maximum(acc[1], 0.0)
    p = (jnp.dot(a0, wout[:HH], preferred_element_type=jnp.float32)
         + jnp.dot(a1, wout[HH:], preferred_element_type=jnp.float32)
         + bout[...])
    out[...] = p


def _dense2(acc, wout, bout):
    n_blocks = NF // R1
    return pl.pallas_call(
        _dense2_body,
        grid=(n_blocks,),
        in_specs=[
            pl.BlockSpec((NCORE, R1, HH), lambda i: (0, i, 0)),
            pl.BlockSpec((H, 1), lambda i: (0, 0)),
            pl.BlockSpec((1,), lambda i: (0,)),
        ],
        out_specs=pl.BlockSpec((R1, 1), lambda i: (i, 0)),
        out_shape=jax.ShapeDtypeStruct((NF, 1), jnp.float32),
    )(acc, wout, bout)


def _edge_chunks(ei_a2f, ei_f2f):
    npad = EPAD - NE
    sa = jnp.concatenate([ei_a2f[0], jnp.zeros((npad,), jnp.int32)])
    da = jnp.concatenate([ei_a2f[1], jnp.full((npad,), NF, jnp.int32)])
    sf = jnp.concatenate([ei_f2f[0] + NF, jnp.zeros((npad,), jnp.int32)])
    df = jnp.concatenate([ei_f2f[1], jnp.full((npad,), NF, jnp.int32)])
    ncp = NCH_PAD - 2 * NCH_REL
    src3 = jnp.concatenate(
        [sa.reshape(NSUB, NCH_REL, K), sf.reshape(NSUB, NCH_REL, K),
         jnp.zeros((NSUB, ncp, K), jnp.int32)], axis=1)
    dst3 = jnp.concatenate(
        [da.reshape(NSUB, NCH_REL, K), df.reshape(NSUB, NCH_REL, K),
         jnp.full((NSUB, ncp, K), NF, jnp.int32)], axis=1)
    return src3.reshape(-1), dst3.reshape(-1)


def kernel(x_flight, x_airport, ei_a2f, ei_f2a, ei_f2f,
           Wp_flight, bp_flight, Wp_airport, bp_airport,
           Wrel_a2f, brel_a2f, Wroot_a2f, broot_a2f,
           Wrel_f2a, brel_f2a, Wroot_f2a, broot_f2a,
           Wrel_f2f, brel_f2f, Wroot_f2f, broot_f2f,
           Wout, bout):
    tab_s, base_s = _dense1(
        x_flight, x_airport, Wp_flight, bp_flight, Wp_airport, bp_airport,
        Wrel_a2f, Wrel_f2f, Wroot_a2f, Wroot_f2f,
        brel_a2f, broot_a2f, brel_f2f, broot_f2f)
    src1, dst1 = _edge_chunks(ei_a2f, ei_f2f)
    acc = _sc_segment_add(
        tab_s.reshape(4 * NF, HH),
        base_s.reshape(NCORE * NF, HH),
        src1, dst1)
    pred = _dense2(acc.reshape(NCORE, NF, HH), Wout, bout)
    return pred.squeeze(-1)


# speedup vs baseline: 1.0021x; 1.0021x over previous
import functools

import jax
import jax.numpy as jnp
from jax import lax
from jax.experimental import pallas as pl
from jax.experimental.pallas import tpu as pltpu
from jax.experimental.pallas import tpu_sc as plsc

NF = 10000
NE = 160000
H = 256
HH = 128
NCORE = 2
NSUB = 16
K = 128
NCH_REL = 79
EPT = NCH_REL * K
EPAD = EPT * NSUB
NCH = 2 * NCH_REL + 2
NCH_PAD = NCH + 4
ROWS_PER_TILE = 632
R1 = 1000


def _dense1_body(xf, xa, wpf, bpf, wpa, bpa, wr1, wr2, wq1, wq2,
                 br1, bq1, br2, bq2, tab_o, base_o):
    hf = jnp.dot(xf[...], wpf[...], preferred_element_type=jnp.float32) + bpf[...]
    ha = jnp.dot(xa[...], wpa[...], preferred_element_type=jnp.float32) + bpa[...]
    tf = jnp.dot(hf, wr2[...], preferred_element_type=jnp.float32) * 0.5
    ta = jnp.dot(ha, wr1[...], preferred_element_type=jnp.float32) * 0.5
    wroot = (wq1[...] + wq2[...]) * 0.5
    bias = (br1[...] + bq1[...] + br2[...] + bq2[...]) * 0.5
    base = jnp.dot(hf, wroot, preferred_element_type=jnp.float32) + bias
    tab_o[0] = ta[:, :HH]
    tab_o[1] = tf[:, :HH]
    tab_o[2] = ta[:, HH:]
    tab_o[3] = tf[:, HH:]
    base_o[0] = base[:, :HH]
    base_o[1] = base[:, HH:]


def _dense1(x_flight, x_airport, wpf, bpf, wpa, bpa, wr1, wr2, wq1, wq2,
            br1, bq1, br2, bq2):
    n_blocks = NF // R1
    full = lambda shape: pl.BlockSpec(shape, lambda i: (0,) * len(shape))
    row_blk = lambda d: pl.BlockSpec((R1, d), lambda i: (i, 0))
    return pl.pallas_call(
        _dense1_body,
        grid=(n_blocks,),
        in_specs=[
            row_blk(256), row_blk(128),
            full((256, H)), full((H,)), full((128, H)), full((H,)),
            full((H, H)), full((H, H)), full((H, H)), full((H, H)),
            full((H,)), full((H,)), full((H,)), full((H,)),
        ],
        out_specs=[
            pl.BlockSpec((4, R1, HH), lambda i: (0, i, 0)),
            pl.BlockSpec((NCORE, R1, HH), lambda i: (0, i, 0)),
        ],
        out_shape=[
            jax.ShapeDtypeStruct((4, NF, HH), jnp.float32),
            jax.ShapeDtypeStruct((NCORE, NF, HH), jnp.float32),
        ],
    )(x_flight, x_airport, wpf, bpf, wpa, bpa, wr1, wr2, wq1, wq2,
      br1, bq1, br2, bq2)


def _sc_segment_add(tab, base2, src1, dst1):
    mesh = plsc.VectorSubcoreMesh(core_axis_name="c", subcore_axis_name="s")

    @functools.partial(
        pl.kernel,
        out_type=jax.ShapeDtypeStruct((NCORE * NF, HH), jnp.float32),
        mesh=mesh,
        scratch_types=[
            pltpu.VMEM_SHARED((NF + 16, HH), jnp.float32),
            [pltpu.VMEM((K,), jnp.int32)] * 2,
            [pltpu.VMEM((K,), jnp.int32)] * 2,
            [pltpu.VMEM((K, HH), jnp.float32)] * 2,
            [pltpu.SemaphoreType.DMA] * 2,
            [pltpu.SemaphoreType.DMA] * 2,
        ],
    )
    def sc_kernel(tab_h, base_h, src_h, dst_h, out_h,
                  acc_s, srcb, dstb, rows, isem, gsem):
        cid = lax.axis_index("c")
        sid = lax.axis_index("s")
        row0 = cid * NF
        win = tab_h.at[pl.ds(pl.multiple_of(cid * (2 * NF), 8), 2 * NF)]
        tile_lo = pl.multiple_of(
            jnp.minimum(sid * ROWS_PER_TILE, NF - ROWS_PER_TILE), 8)

        def coff(c):
            return pl.multiple_of((sid * NCH_PAD + c) * K, K)

        def idx_start(c, s):
            pltpu.async_copy(src_h.at[pl.ds(coff(c), K)], srcb[s], isem[s])
            pltpu.async_copy(dst_h.at[pl.ds(coff(c), K)], dstb[s], isem[s])

        def idx_wait(s):
            pltpu.make_async_copy(src_h.at[pl.ds(0, K)], srcb[s], isem[s]).wait()
            pltpu.make_async_copy(dst_h.at[pl.ds(0, K)], dstb[s], isem[s]).wait()

        def gather_start(s):
            pltpu.async_copy(win.at[srcb[s]], rows[s], gsem[s])

        def gather_wait(s):
            pltpu.make_async_copy(win.at[srcb[0]], rows[s], gsem[s]).wait()

        def scatter(s):
            pltpu.sync_copy(rows[s], acc_s.at[dstb[s]], add=True)

        pltpu.sync_copy(base_h.at[pl.ds(row0 + tile_lo, ROWS_PER_TILE)],
                        acc_s.at[pl.ds(tile_lo, ROWS_PER_TILE)])
        idx_start(0, 0)
        idx_start(1, 1)
        idx_wait(0)
        gather_start(0)
        plsc.subcore_barrier()

        def pair(i, carry):
            gather_wait(0)
            idx_wait(1)
            gather_start(1)
            scatter(0)
            idx_start(i + 2, 0)
            gather_wait(1)
            idx_wait(0)
            gather_start(0)
            scatter(1)
            idx_start(i + 3, 1)
            return carry

        lax.fori_loop(0, NCH // 2, lambda i, c: pair(2 * i, c), 0)
        gather_wait(0)
        idx_wait(1)
        plsc.subcore_barrier()
        pltpu.sync_copy(acc_s.at[pl.ds(tile_lo, ROWS_PER_TILE)],
                        out_h.at[pl.ds(row0 + tile_lo, ROWS_PER_TILE)])

    return sc_kernel(tab, base2, src1, dst1)


def _dense2_body(acc, wout, bout, out):
    a0 = jnp.maximum(acc[0], 0.0)
    a1 = jnp.maximum(acc[1], 0.0)
    p = (jnp.dot(a0, wout[:HH], preferred_element_type=jnp.float32)
         + jnp.dot(a1, wout[HH:], preferred_element_type=jnp.float32)
         + bout[...])
    out[...] = p


def _dense2(acc, wout, bout):
    n_blocks = NF // R1
    return pl.pallas_call(
        _dense2_body,
        grid=(n_blocks,),
        in_specs=[
            pl.BlockSpec((NCORE, R1, HH), lambda i: (0, i, 0)),
            pl.BlockSpec((H, 1), lambda i: (0, 0)),
            pl.BlockSpec((1,), lambda i: (0,)),
        ],
        out_specs=pl.BlockSpec((R1, 1), lambda i: (i, 0)),
        out_shape=jax.ShapeDtypeStruct((NF, 1), jnp.float32),
    )(acc, wout, bout)


def _edge_chunks(ei_a2f, ei_f2f):
    npad = EPAD - NE
    sa = jnp.concatenate([ei_a2f[0], jnp.zeros((npad,), jnp.int32)])
    da = jnp.concatenate([ei_a2f[1], jnp.full((npad,), NF, jnp.int32)])
    sf = jnp.concatenate([ei_f2f[0] + NF, jnp.zeros((npad,), jnp.int32)])
    df = jnp.concatenate([ei_f2f[1], jnp.full((npad,), NF, jnp.int32)])
    ncp = NCH_PAD - 2 * NCH_REL
    src3 = jnp.concatenate(
        [sa.reshape(NSUB, NCH_REL, K), sf.reshape(NSUB, NCH_REL, K),
         jnp.zeros((NSUB, ncp, K), jnp.int32)], axis=1)
    dst3 = jnp.concatenate(
        [da.reshape(NSUB, NCH_REL, K), df.reshape(NSUB, NCH_REL, K),
         jnp.full((NSUB, ncp, K), NF, jnp.int32)], axis=1)
    return src3.reshape(-1), dst3.reshape(-1)


def kernel(x_flight, x_airport, ei_a2f, ei_f2a, ei_f2f,
           Wp_flight, bp_flight, Wp_airport, bp_airport,
           Wrel_a2f, brel_a2f, Wroot_a2f, broot_a2f,
           Wrel_f2a, brel_f2a, Wroot_f2a, broot_f2a,
           Wrel_f2f, brel_f2f, Wroot_f2f, broot_f2f,
           Wout, bout):
    tab_s, base_s = _dense1(
        x_flight, x_airport, Wp_flight, bp_flight, Wp_airport, bp_airport,
        Wrel_a2f, Wrel_f2f, Wroot_a2f, Wroot_f2f,
        brel_a2f, broot_a2f, brel_f2f, broot_f2f)
    src1, dst1 = _edge_chunks(ei_a2f, ei_f2f)
    acc = _sc_segment_add(
        tab_s.reshape(4 * NF, HH),
        base_s.reshape(NCORE * NF, HH),
        src1, dst1)
    pred = _dense2(acc.reshape(NCORE, NF, HH), Wout, bout)
    return pred.squeeze(-1)


# speedup vs baseline: 1.3384x; 1.3356x over previous
import functools

import jax
import jax.numpy as jnp
from jax import lax
from jax.experimental import pallas as pl
from jax.experimental.pallas import tpu as pltpu
from jax.experimental.pallas import tpu_sc as plsc

NF = 10000
NE = 160000
H = 256
HH = 128
NCORE = 2
NSUB = 16
K = 128
NCH_REL = 79
EPT = NCH_REL * K
EPAD = EPT * NSUB
NCH = 2 * NCH_REL
NCH_PAD = NCH + 2
ROWS_PER_TILE = 632
R1 = 1000


def _dense1_body(xf, xa, wpf, bpf, wpa, bpa, wr1, wr2, wq1, wq2,
                 br1, bq1, br2, bq2, tab_o, base_o):
    hf = jnp.dot(xf[...], wpf[...], preferred_element_type=jnp.float32) + bpf[...]
    ha = jnp.dot(xa[...], wpa[...], preferred_element_type=jnp.float32) + bpa[...]
    tf = jnp.dot(hf, wr2[...], preferred_element_type=jnp.float32) * 0.5
    ta = jnp.dot(ha, wr1[...], preferred_element_type=jnp.float32) * 0.5
    wroot = (wq1[...] + wq2[...]) * 0.5
    bias = (br1[...] + bq1[...] + br2[...] + bq2[...]) * 0.5
    base = jnp.dot(hf, wroot, preferred_element_type=jnp.float32) + bias
    tab_o[0] = ta[:, :HH]
    tab_o[1] = tf[:, :HH]
    tab_o[2] = ta[:, HH:]
    tab_o[3] = tf[:, HH:]
    base_o[0] = base[:, :HH]
    base_o[1] = base[:, HH:]


def _dense1(x_flight, x_airport, wpf, bpf, wpa, bpa, wr1, wr2, wq1, wq2,
            br1, bq1, br2, bq2):
    n_blocks = NF // R1
    full = lambda shape: pl.BlockSpec(shape, lambda i: (0,) * len(shape))
    row_blk = lambda d: pl.BlockSpec((R1, d), lambda i: (i, 0))
    return pl.pallas_call(
        _dense1_body,
        grid=(n_blocks,),
        in_specs=[
            row_blk(256), row_blk(128),
            full((256, H)), full((H,)), full((128, H)), full((H,)),
            full((H, H)), full((H, H)), full((H, H)), full((H, H)),
            full((H,)), full((H,)), full((H,)), full((H,)),
        ],
        out_specs=[
            pl.BlockSpec((4, R1, HH), lambda i: (0, i, 0)),
            pl.BlockSpec((NCORE, R1, HH), lambda i: (0, i, 0)),
        ],
        out_shape=[
            jax.ShapeDtypeStruct((4, NF, HH), jnp.float32),
            jax.ShapeDtypeStruct((NCORE, NF, HH), jnp.float32),
        ],
    )(x_flight, x_airport, wpf, bpf, wpa, bpa, wr1, wr2, wq1, wq2,
      br1, bq1, br2, bq2)


def _sc_segment_add(tab, base2, src1, dst1):
    mesh = plsc.VectorSubcoreMesh(core_axis_name="c", subcore_axis_name="s")

    @functools.partial(
        pl.kernel,
        out_type=jax.ShapeDtypeStruct((NCORE * NF, HH), jnp.float32),
        mesh=mesh,
        scratch_types=[
            pltpu.VMEM_SHARED((NF + 16, HH), jnp.float32),
            [pltpu.VMEM((K,), jnp.int32)] * 2,
            [pltpu.VMEM((K,), jnp.int32)] * 2,
            [pltpu.VMEM((K, HH), jnp.float32)] * 2,
            [pltpu.SemaphoreType.DMA] * 2,
            [pltpu.SemaphoreType.DMA] * 2,
        ],
    )
    def sc_kernel(tab_h, base_h, src_h, dst_h, out_h,
                  acc_s, srcb, dstb, rows, isem, gsem):
        cid = lax.axis_index("c")
        sid = lax.axis_index("s")
        row0 = cid * NF
        win = tab_h.at[pl.ds(pl.multiple_of(cid * (2 * NF), 8), 2 * NF)]
        tile_lo = pl.multiple_of(
            jnp.minimum(sid * ROWS_PER_TILE, NF - ROWS_PER_TILE), 8)

        def coff(c):
            return pl.multiple_of((sid * NCH_PAD + c) * K, K)

        def idx_start(c, s):
            pltpu.async_copy(src_h.at[pl.ds(coff(c), K)], srcb[s], isem[s])
            pltpu.async_copy(dst_h.at[pl.ds(coff(c), K)], dstb[s], isem[s])

        def idx_wait(s):
            pltpu.make_async_copy(src_h.at[pl.ds(0, K)], srcb[s], isem[s]).wait()
            pltpu.make_async_copy(dst_h.at[pl.ds(0, K)], dstb[s], isem[s]).wait()

        def gather_start(s):
            pltpu.async_copy(win.at[srcb[s]], rows[s], gsem[s])

        def gather_wait(s):
            pltpu.make_async_copy(win.at[srcb[0]], rows[s], gsem[s]).wait()

        def scatter(s):
            pltpu.sync_copy(rows[s], acc_s.at[dstb[s]], add=True)

        pltpu.sync_copy(base_h.at[pl.ds(row0 + tile_lo, ROWS_PER_TILE)],
                        acc_s.at[pl.ds(tile_lo, ROWS_PER_TILE)])
        idx_start(0, 0)
        idx_start(1, 1)
        idx_wait(0)
        gather_start(0)
        plsc.subcore_barrier()

        def pair(i, carry):
            gather_wait(0)
            idx_wait(1)
            gather_start(1)
            scatter(0)
            idx_start(i + 2, 0)
            gather_wait(1)
            idx_wait(0)
            gather_start(0)
            scatter(1)
            idx_start(i + 3, 1)
            return carry

        lax.fori_loop(0, NCH // 2, lambda i, c: pair(2 * i, c), 0)
        gather_wait(0)
        idx_wait(1)
        plsc.subcore_barrier()
        pltpu.sync_copy(acc_s.at[pl.ds(tile_lo, ROWS_PER_TILE)],
                        out_h.at[pl.ds(row0 + tile_lo, ROWS_PER_TILE)])

    return sc_kernel(tab, base2, src1, dst1)


def _dense2_body(acc, wout, bout, out):
    a0 = jnp.maximum(acc[0], 0.0)
    a1 = jnp.maximum(acc[1], 0.0)
    p = (jnp.dot(a0, wout[:HH], preferred_element_type=jnp.float32)
         + jnp.dot(a1, wout[HH:], preferred_element_type=jnp.float32)
         + bout[...])
    out[...] = p


def _dense2(acc, wout, bout):
    n_blocks = NF // R1
    return pl.pallas_call(
        _dense2_body,
        grid=(n_blocks,),
        in_specs=[
            pl.BlockSpec((NCORE, R1, HH), lambda i: (0, i, 0)),
            pl.BlockSpec((H, 1), lambda i: (0, 0)),
            pl.BlockSpec((1,), lambda i: (0,)),
        ],
        out_specs=pl.BlockSpec((R1, 1), lambda i: (i, 0)),
        out_shape=jax.ShapeDtypeStruct((NF, 1), jnp.float32),
    )(acc, wout, bout)


def _edge_chunks(ei_a2f, ei_f2f):
    npad = EPAD - NE
    sa = jnp.concatenate([ei_a2f[0], jnp.zeros((npad,), jnp.int32)])
    da = jnp.concatenate([ei_a2f[1], jnp.full((npad,), NF, jnp.int32)])
    sf = jnp.concatenate([ei_f2f[0] + NF, jnp.zeros((npad,), jnp.int32)])
    df = jnp.concatenate([ei_f2f[1], jnp.full((npad,), NF, jnp.int32)])
    ncp = NCH_PAD - 2 * NCH_REL
    src3 = jnp.concatenate(
        [sa.reshape(NSUB, NCH_REL, K), sf.reshape(NSUB, NCH_REL, K),
         jnp.zeros((NSUB, ncp, K), jnp.int32)], axis=1)
    dst3 = jnp.concatenate(
        [da.reshape(NSUB, NCH_REL, K), df.reshape(NSUB, NCH_REL, K),
         jnp.full((NSUB, ncp, K), NF, jnp.int32)], axis=1)
    return src3.reshape(-1), dst3.reshape(-1)


def kernel(x_flight, x_airport, ei_a2f, ei_f2a, ei_f2f,
           Wp_flight, bp_flight, Wp_airport, bp_airport,
           Wrel_a2f, brel_a2f, Wroot_a2f, broot_a2f,
           Wrel_f2a, brel_f2a, Wroot_f2a, broot_f2a,
           Wrel_f2f, brel_f2f, Wroot_f2f, broot_f2f,
           Wout, bout):
    tab_s, base_s = _dense1(
        x_flight, x_airport, Wp_flight, bp_flight, Wp_airport, bp_airport,
        Wrel_a2f, Wrel_f2f, Wroot_a2f, Wroot_f2f,
        brel_a2f, broot_a2f, brel_f2f, broot_f2f)
    src1, dst1 = _edge_chunks(ei_a2f, ei_f2f)
    acc = _sc_segment_add(
        tab_s.reshape(4 * NF, HH),
        base_s.reshape(NCORE * NF, HH),
        src1, dst1)
    pred = _dense2(acc.reshape(NCORE, NF, HH), Wout, bout)
    return pred.squeeze(-1)
